# baseline (speedup 1.0000x reference)
import functools

import jax
import jax.numpy as jnp
from jax import lax
from jax.experimental import pallas as pl
from jax.experimental.pallas import tpu as pltpu
from jax.experimental.pallas import tpu_sc as plsc

B = 4
S = 8192
S1 = S + 1
D = 1024
K = int(S * (1 - 0.1))
P = K + 1
R = 64
L = 128
NBITS = 13

WPB = 8
WSZ = 928
CW = 32
NCW = 29
NB = 3
NG = 2
WLEN = WSZ

_sc_mesh = plsc.VectorSubcoreMesh(core_axis_name="c", subcore_axis_name="s")


def _sort_body(m_ref, out_ref):
    key = m_ref[0]
    rows = lax.broadcasted_iota(jnp.int32, (R, L), 0)
    lanes = lax.broadcasted_iota(jnp.int32, (R, L), 1)
    e = rows * L + lanes
    idx = e
    mcache = {}

    def bitmask(bit):
        if bit not in mcache:
            if bit >= S:
                mcache[bit] = None
            elif bit < L:
                mcache[bit] = (lanes & bit) == 0
            else:
                mcache[bit] = (rows & (bit >> 7)) == 0
        return mcache[bit]

    for kb in range(1, NBITS + 1):
        fwd = bitmask(1 << kb)
        for jb in range(kb - 1, -1, -1):
            d = 1 << jb
            lo = bitmask(d)
            ax, sh, n = (1, d, L) if d < L else (0, d >> 7, R)
            pk = jnp.where(lo, pltpu.roll(key, n - sh, ax),
                           pltpu.roll(key, sh, ax))
            pi = jnp.where(lo, pltpu.roll(idx, n - sh, ax),
                           pltpu.roll(idx, sh, ax))
            mb = (key > pk) | ((key == pk) & (idx < pi))
            x1 = jnp.logical_xor(mb, lo)
            keep = jnp.logical_not(x1) if fwd is None \
                else jnp.logical_xor(x1, fwd)
            key = jnp.where(keep, key, pk)
            idx = jnp.where(keep, idx, pi)

    rolled = pltpu.roll(idx, 1, 1)
    rowr = pltpu.roll(rolled, 1, 0)
    shifted = jnp.where(lanes == 0, rowr, rolled)
    out_ref[0] = jnp.where(e == 0, 0, shifted + 1)


_sort = pl.pallas_call(
    _sort_body,
    grid=(B,),
    in_specs=[pl.BlockSpec((1, R, L), lambda b: (b, 0, 0))],
    out_specs=pl.BlockSpec((1, R, L), lambda b: (b, 0, 0)),
    out_shape=jax.ShapeDtypeStruct((B, R, L), jnp.int32),
)


@functools.partial(
    pl.kernel,
    out_type=jax.ShapeDtypeStruct((B, P, D), jnp.float32),
    mesh=_sc_mesh,
    compiler_params=pltpu.CompilerParams(needs_layout_passes=False),
    scratch_types=[
        pltpu.VMEM((WLEN,), jnp.int32),
        pltpu.VMEM((NB, CW, D), jnp.float32),
        pltpu.SemaphoreType.DMA((NB,)),
        pltpu.SemaphoreType.DMA((NB,)),
    ],
)
def _topk_gather(g_hbm, x_hbm, out_hbm, gwin_v, rows_v, isems, osems):
    wid = lax.axis_index("s") * 2 + lax.axis_index("c")
    b = wid // WPB
    wi = lax.rem(wid, WPB)
    r0 = wi * WSZ
    r1 = jnp.minimum(r0 + WSZ, P)
    pltpu.sync_copy(g_hbm.at[pl.ds((b * 1024 + wi * 116) * 8, WLEN)], gwin_v)
    lane = lax.broadcasted_iota(jnp.int32, (16,), 0)
    bnd8 = (r1 - CW) // 8

    def chunk8(c):
        return jnp.minimum(wi * (WSZ // 8) + c * (CW // 8), bnd8)

    def start_g(c):
        idx = gwin_v.at[pl.ds((chunk8(c) - wi * (WSZ // 8)) * 8, CW)]
        return pltpu.async_copy(x_hbm.at[b].at[idx], rows_v.at[c % NB],
                                isems.at[c % NB])

    gd = [None] * NCW
    od = [None] * NCW
    for c in range(NG):
        gd[c] = start_g(c)
    for c in range(NCW):
        gd[c].wait()
        od[c] = pltpu.async_copy(
            rows_v.at[c % NB],
            out_hbm.at[b].at[pl.ds(chunk8(c) * 8, CW)],
            osems.at[c % NB])
        n = c + NG
        if n < NCW:
            if n >= NB:
                od[n - NB].wait()
            gd[n] = start_g(n)
    for c in range(NCW - NB, NCW):
        od[c].wait()

    bt = r1 - CW
    g0 = pltpu.async_copy(x_hbm.at[b].at[gwin_v[pl.ds(bt - r0, 16)]],
                          rows_v.at[0].at[pl.ds(0, 16)], isems.at[0])
    g1 = pltpu.async_copy(x_hbm.at[b].at[gwin_v[pl.ds(bt - r0 + 16, 16)]],
                          rows_v.at[0].at[pl.ds(16, 16)], isems.at[1])
    g0.wait()
    g1.wait()
    t0 = pltpu.async_copy(rows_v.at[0].at[pl.ds(0, 16)],
                          out_hbm.at[b].at[bt + lane], osems.at[0])
    t1 = pltpu.async_copy(rows_v.at[0].at[pl.ds(16, 16)],
                          out_hbm.at[b].at[bt + 16 + lane], osems.at[1])
    t0.wait()
    t1.wait()


def kernel(x, mask):
    g = _sort(mask.reshape(B, R, L))
    return _topk_gather(g.reshape(B * S), x)


# speedup vs baseline: 1.0336x; 1.0336x over previous
import functools

import jax
import jax.numpy as jnp
from jax import lax
from jax.experimental import pallas as pl
from jax.experimental.pallas import tpu as pltpu
from jax.experimental.pallas import tpu_sc as plsc

B = 4
S = 8192
S1 = S + 1
D = 1024
K = int(S * (1 - 0.1))
P = K + 1
R = 64
L = 128
NBITS = 13

WPB = 8
WSZ = 928
CW = 32
NCW = 29
NB = 3
NG = 2
WLEN = WSZ

_sc_mesh = plsc.VectorSubcoreMesh(core_axis_name="c", subcore_axis_name="s")


BPP = 4


def _sort_body(m_ref, out_ref):
    key = m_ref[...]
    rows = lax.broadcasted_iota(jnp.int32, (1, R, L), 1)
    lanes = lax.broadcasted_iota(jnp.int32, (1, R, L), 2)
    e = rows * L + lanes
    idx = jnp.broadcast_to(e, (BPP, R, L))
    mcache = {}

    def bitmask(bit):
        if bit not in mcache:
            if bit >= S:
                mcache[bit] = None
            elif bit < L:
                mcache[bit] = (lanes & bit) == 0
            else:
                mcache[bit] = (rows & (bit >> 7)) == 0
        return mcache[bit]

    for kb in range(1, NBITS + 1):
        fwd = bitmask(1 << kb)
        for jb in range(kb - 1, -1, -1):
            d = 1 << jb
            lo = bitmask(d)
            ax, sh, n = (2, d, L) if d < L else (1, d >> 7, R)
            pk = jnp.where(lo, pltpu.roll(key, n - sh, ax),
                           pltpu.roll(key, sh, ax))
            pi = jnp.where(lo, pltpu.roll(idx, n - sh, ax),
                           pltpu.roll(idx, sh, ax))
            mb = (key > pk) | ((key == pk) & (idx < pi))
            x1 = jnp.logical_xor(mb, lo)
            keep = jnp.logical_not(x1) if fwd is None \
                else jnp.logical_xor(x1, fwd)
            key = jnp.where(keep, key, pk)
            idx = jnp.where(keep, idx, pi)

    rolled = pltpu.roll(idx, 1, 2)
    rowr = pltpu.roll(rolled, 1, 1)
    shifted = jnp.where(lanes == 0, rowr, rolled)
    out_ref[...] = jnp.where(e == 0, 0, shifted + 1)


_sort = pl.pallas_call(
    _sort_body,
    grid=(B // BPP,),
    in_specs=[pl.BlockSpec((BPP, R, L), lambda g: (g, 0, 0))],
    out_specs=pl.BlockSpec((BPP, R, L), lambda g: (g, 0, 0)),
    out_shape=jax.ShapeDtypeStruct((B, R, L), jnp.int32),
)


@functools.partial(
    pl.kernel,
    out_type=jax.ShapeDtypeStruct((B, P, D), jnp.float32),
    mesh=_sc_mesh,
    compiler_params=pltpu.CompilerParams(needs_layout_passes=False),
    scratch_types=[
        pltpu.VMEM((WLEN,), jnp.int32),
        pltpu.VMEM((NB, CW, D), jnp.float32),
        pltpu.SemaphoreType.DMA((NB,)),
        pltpu.SemaphoreType.DMA((NB,)),
    ],
)
def _topk_gather(g_hbm, x_hbm, out_hbm, gwin_v, rows_v, isems, osems):
    wid = lax.axis_index("s") * 2 + lax.axis_index("c")
    b = wid // WPB
    wi = lax.rem(wid, WPB)
    r0 = wi * WSZ
    r1 = jnp.minimum(r0 + WSZ, P)
    pltpu.sync_copy(g_hbm.at[pl.ds((b * 1024 + wi * 116) * 8, WLEN)], gwin_v)
    lane = lax.broadcasted_iota(jnp.int32, (16,), 0)
    bnd8 = (r1 - CW) // 8

    def chunk8(c):
        return jnp.minimum(wi * (WSZ // 8) + c * (CW // 8), bnd8)

    def start_g(c):
        idx = gwin_v.at[pl.ds((chunk8(c) - wi * (WSZ // 8)) * 8, CW)]
        return pltpu.async_copy(x_hbm.at[b].at[idx], rows_v.at[c % NB],
                                isems.at[c % NB])

    gd = [None] * NCW
    od = [None] * NCW
    for c in range(NG):
        gd[c] = start_g(c)
    for c in range(NCW):
        gd[c].wait()
        od[c] = pltpu.async_copy(
            rows_v.at[c % NB],
            out_hbm.at[b].at[pl.ds(chunk8(c) * 8, CW)],
            osems.at[c % NB])
        n = c + NG
        if n < NCW:
            if n >= NB:
                od[n - NB].wait()
            gd[n] = start_g(n)
    for c in range(NCW - NB, NCW):
        od[c].wait()

    bt = r1 - CW
    g0 = pltpu.async_copy(x_hbm.at[b].at[gwin_v[pl.ds(bt - r0, 16)]],
                          rows_v.at[0].at[pl.ds(0, 16)], isems.at[0])
    g1 = pltpu.async_copy(x_hbm.at[b].at[gwin_v[pl.ds(bt - r0 + 16, 16)]],
                          rows_v.at[0].at[pl.ds(16, 16)], isems.at[1])
    g0.wait()
    g1.wait()
    t0 = pltpu.async_copy(rows_v.at[0].at[pl.ds(0, 16)],
                          out_hbm.at[b].at[bt + lane], osems.at[0])
    t1 = pltpu.async_copy(rows_v.at[0].at[pl.ds(16, 16)],
                          out_hbm.at[b].at[bt + 16 + lane], osems.at[1])
    t0.wait()
    t1.wait()


def kernel(x, mask):
    g = _sort(mask.reshape(B, R, L))
    return _topk_gather(g.reshape(B * S), x)
